# Initial kernel scaffold; baseline (speedup 1.0000x reference)
import jax
import jax.numpy as jnp
from jax.experimental import pallas as pl


def kernel(x, node_attr, mask_downstream_adj, mask_khop_up_adj, full_path_edge_attr_adj, outlet_index, params):
    raise NotImplementedError("write your pallas kernel here")


# speedup vs baseline: 1157.1814x; 1157.1814x over previous
import functools

import jax
import jax.numpy as jnp
from jax.experimental import pallas as pl
from jax.experimental.pallas import tpu as pltpu

_P_STEPS = 12
_TAIL = 6


def _prep_kernel(attr_ref, md_ref, mk_ref, wpe1_ref, bpe1_ref, wpe2_ref,
                 bpe2_ref, w_ref):
    attr = attr_ref[0]
    EA, R, N = attr.shape
    pe1 = jnp.tanh(
        jax.lax.dot_general(wpe1_ref[...], attr.reshape(EA, R * N),
                            (((0,), (0,)), ((), ())),
                            preferred_element_type=jnp.float32)
        + bpe1_ref[...])
    pe = jax.lax.dot_general(wpe2_ref[...], pe1, (((1,), (0,)), ((), ())),
                             preferred_element_type=jnp.float32)
    pe = pe.reshape(R, N) + bpe2_ref[0, 0]
    m = jnp.clip(md_ref[0] + mk_ref[0], 0.0, 1.0)
    w_ref[0] = jax.nn.sigmoid(pe) * m


def _recur_kernel(T, L, xt_ref, nattr_ref, w_ref, outlet_ref,
                  win_ref, bin_ref, wfilm_ref, bfilm_ref,
                  enc_wx_refs, enc_wh_refs, enc_b_refs,
                  dec_wx_refs, dec_wh_refs, dec_b_refs,
                  wout_ref, bout_ref, wfb_ref,
                  out_ref, hproj_ref, pred_ref):
    w = w_ref[0]
    N = w.shape[0]
    H = win_ref.shape[1]
    inv_deg = 1.0 / (jnp.sum(w, axis=0) + 1e-6)

    xt = xt_ref[0]
    F = xt.shape[2]
    hp = jax.lax.dot_general(xt.reshape(T * N, F), win_ref[...],
                             (((1,), (0,)), ((), ())),
                             preferred_element_type=jnp.float32) + bin_ref[...]
    film = jax.lax.dot_general(nattr_ref[0], wfilm_ref[...],
                               (((1,), (0,)), ((), ())),
                               preferred_element_type=jnp.float32) + bfilm_ref[...]
    hp = hp.reshape(T, N, H)
    hproj_ref[...] = hp * (1.0 + film[None, :, :H]) + film[None, :, H:]

    def gru(inp, h, wx_ref, wh_ref, b_ref):
        agg = jax.lax.dot_general(w, h, (((0,), (0,)), ((), ())),
                                  preferred_element_type=jnp.float32)
        agg = agg * inv_deg[:, None]
        gx = jax.lax.dot_general(inp, wx_ref[...], (((1,), (0,)), ((), ())),
                                 preferred_element_type=jnp.float32) + b_ref[...]
        gh = jax.lax.dot_general(agg, wh_ref[...], (((1,), (0,)), ((), ())),
                                 preferred_element_type=jnp.float32)
        r = jax.nn.sigmoid(gx[:, :H] + gh[:, :H])
        z = jax.nn.sigmoid(gx[:, H:2 * H] + gh[:, H:2 * H])
        n = jnp.tanh(gx[:, 2 * H:] + r * gh[:, 2 * H:])
        return (1.0 - z) * n + z * h

    zeros = jnp.zeros((N, H), jnp.float32)

    def enc_body(t, carry):
        hs, acc = carry
        inp = hproj_ref[t]
        new_hs = []
        for l in range(L):
            inp = gru(inp, hs[l], enc_wx_refs[l], enc_wh_refs[l], enc_b_refs[l])
            new_hs.append(inp)
        acc = acc + jnp.where(t >= T - _TAIL, 1.0, 0.0) * new_hs[-1]
        return tuple(new_hs), acc

    hs, acc = jax.lax.fori_loop(0, T, enc_body, ((zeros,) * L, zeros))
    context = acc * (1.0 / _TAIL)

    wfb = wfb_ref[...]
    wout = wout_ref[...]
    bout = bout_ref[0, 0]

    def dec_body(p, carry):
        hs, prev_y = carry
        inp = context + prev_y * wfb
        new_hs = []
        for l in range(L):
            inp = gru(inp, hs[l], dec_wx_refs[l], dec_wh_refs[l], dec_b_refs[l])
            new_hs.append(inp)
        y = jnp.sum(new_hs[-1] * wout, axis=1, keepdims=True) + bout
        pred_ref[p, :] = y[:, 0]
        return tuple(new_hs), y

    jax.lax.fori_loop(0, _P_STEPS, dec_body, (hs, jnp.zeros((N, 1), jnp.float32)))

    outlet = outlet_ref[0, 0]
    K = outlet.shape[0]
    iota = jax.lax.broadcasted_iota(jnp.int32, (K, N), 1)
    onehot = (iota == outlet[:, None]).astype(jnp.float32)
    out_ref[0] = jax.lax.dot_general(pred_ref[...], onehot,
                                     (((1,), (1,)), ((), ())),
                                     preferred_element_type=jnp.float32)


def kernel(x, node_attr, mask_downstream_adj, mask_khop_up_adj,
           full_path_edge_attr_adj, outlet_index, params):
    B, N, T, F = x.shape
    NA = node_attr.shape[-1]
    EA = full_path_edge_attr_adj.shape[-1]
    PH = params["W_pe1"].shape[1]
    H = params["W_in"].shape[1]
    K = outlet_index.shape[-1]
    L = sum(1 for k in params if k.startswith("enc_Wx_"))

    f32 = jnp.float32
    R = 64
    attr_t = jnp.transpose(full_path_edge_attr_adj, (0, 3, 1, 2))
    w = pl.pallas_call(
        _prep_kernel,
        grid=(B, N // R),
        in_specs=[
            pl.BlockSpec((1, EA, R, N), lambda b, r: (b, 0, r, 0)),
            pl.BlockSpec((1, R, N), lambda b, r: (b, r, 0)),
            pl.BlockSpec((1, R, N), lambda b, r: (b, r, 0)),
            pl.BlockSpec((EA, PH), lambda b, r: (0, 0)),
            pl.BlockSpec((PH, 1), lambda b, r: (0, 0)),
            pl.BlockSpec((1, PH), lambda b, r: (0, 0)),
            pl.BlockSpec((1, 1), lambda b, r: (0, 0)),
        ],
        out_specs=pl.BlockSpec((1, R, N), lambda b, r: (b, r, 0)),
        out_shape=jax.ShapeDtypeStruct((B, N, N), f32),
    )(attr_t, mask_downstream_adj, mask_khop_up_adj,
      params["W_pe1"], params["b_pe1"].reshape(PH, 1),
      params["W_pe2"].reshape(1, PH), params["b_pe2"].reshape(1, 1))

    xt = jnp.transpose(x, (0, 2, 1, 3))
    outlet3 = outlet_index.reshape(B, 1, K)

    full = lambda shape: pl.BlockSpec(shape, lambda b: (0,) * len(shape))
    weight_ops = [
        params["W_in"], params["b_in"].reshape(1, H),
        params["W_film"], params["b_film"].reshape(1, 2 * H),
    ]
    weight_specs = [full((F, H)), full((1, H)),
                    full((NA, 2 * H)), full((1, 2 * H))]
    for tag in ("enc", "dec"):
        for nm in ("Wx", "Wh", "b"):
            for l in range(L):
                a = params[f"{tag}_{nm}_{l}"]
                if a.ndim == 1:
                    a = a.reshape(1, 3 * H)
                weight_ops.append(a)
                weight_specs.append(full(a.shape))
    weight_ops += [params["W_out"].reshape(1, H), params["b_out"].reshape(1, 1),
                   params["W_fb"]]
    weight_specs += [full((1, H)), full((1, 1)), full((1, H))]

    def body(*refs):
        xt_ref, nattr_ref, w_ref, outlet_ref = refs[:4]
        wr = list(refs[4:4 + len(weight_ops)])
        out_ref, hproj_ref, pred_ref = refs[4 + len(weight_ops):]
        win_ref, bin_ref, wfilm_ref, bfilm_ref = wr[:4]
        idx = 4
        groups = {}
        for tag in ("enc", "dec"):
            for nm in ("Wx", "Wh", "b"):
                groups[(tag, nm)] = wr[idx:idx + L]
                idx += L
        wout_ref, bout_ref, wfb_ref = wr[idx:idx + 3]
        _recur_kernel(T, L, xt_ref, nattr_ref, w_ref, outlet_ref,
                      win_ref, bin_ref, wfilm_ref, bfilm_ref,
                      groups[("enc", "Wx")], groups[("enc", "Wh")],
                      groups[("enc", "b")],
                      groups[("dec", "Wx")], groups[("dec", "Wh")],
                      groups[("dec", "b")],
                      wout_ref, bout_ref, wfb_ref,
                      out_ref, hproj_ref, pred_ref)

    out = pl.pallas_call(
        body,
        grid=(B,),
        in_specs=[
            pl.BlockSpec((1, T, N, F), lambda b: (b, 0, 0, 0)),
            pl.BlockSpec((1, N, NA), lambda b: (b, 0, 0)),
            pl.BlockSpec((1, N, N), lambda b: (b, 0, 0)),
            pl.BlockSpec((1, 1, K), lambda b: (b, 0, 0)),
        ] + weight_specs,
        out_specs=pl.BlockSpec((1, _P_STEPS, K), lambda b: (b, 0, 0)),
        out_shape=jax.ShapeDtypeStruct((B, _P_STEPS, K), f32),
        scratch_shapes=[pltpu.VMEM((T, N, H), f32),
                        pltpu.VMEM((_P_STEPS, N), f32)],
    )(xt, node_attr, w, outlet3, *weight_ops)
    return out
